# baseline (speedup 1.0000x reference)
import jax
import jax.numpy as jnp
from jax import lax
from jax.experimental import pallas as pl
from jax.experimental.pallas import tpu as pltpu

_N, _G = 100000, 512
_BN = 800
_NBUF = 5
_S = _N // _BN
_OUTER = _S // _NBUF


def _body(pred_hbm, tgt_hbm, obs_hbm, msk_hbm, out_ref,
          pbuf, tbuf, obuf, mbuf, eacc_ref, macc_ref, sems):
    def copies(s, b):
        sl = pl.ds(s * _BN, _BN)
        return (
            pltpu.make_async_copy(pred_hbm.at[sl], pbuf.at[b], sems.at[b, 0]),
            pltpu.make_async_copy(tgt_hbm.at[sl], tbuf.at[b], sems.at[b, 1]),
            pltpu.make_async_copy(obs_hbm.at[sl], obuf.at[b], sems.at[b, 2]),
            pltpu.make_async_copy(msk_hbm.at[sl], mbuf.at[b], sems.at[b, 3]),
        )

    for b in range(_NBUF):
        for c in copies(b, b):
            c.start()

    eacc_ref[...] = jnp.zeros((8, _G), jnp.float32)
    macc_ref[...] = jnp.zeros((8, _G), jnp.float32)

    def outer(g, carry):
        for j in range(_NBUF):
            s = g * _NBUF + j
            for c in copies(s, j):
                c.wait()
            p = pbuf[j]
            t = tbuf[j]
            o = obuf[j]
            mi = mbuf[j]
            m = jnp.where(o > 0.5, mi.astype(jnp.float32), 0.0)
            d = p - t
            e = d * d * m
            eacc_ref[...] += jnp.sum(e.reshape(_BN // 8, 8, _G), axis=0)
            macc_ref[...] += jnp.sum(m.reshape(_BN // 8, 8, _G), axis=0)

            @pl.when(s + _NBUF < _S)
            def _():
                for c in copies(s + _NBUF, j):
                    c.start()
        return carry

    lax.fori_loop(0, _OUTER, outer, 0)
    ssum = jnp.sum(eacc_ref[...])
    csum = jnp.sum(macc_ref[...])
    out_ref[0, 0] = ssum / jnp.maximum(csum, 1.0)


def _masked_mse(pred_expr, target_expr, obs_mask, mask_u8, interpret=False):
    out = pl.pallas_call(
        _body,
        in_specs=[pl.BlockSpec(memory_space=pl.ANY)] * 4,
        out_specs=pl.BlockSpec(memory_space=pltpu.SMEM),
        out_shape=jax.ShapeDtypeStruct((1, 1), jnp.float32),
        scratch_shapes=[
            pltpu.VMEM((_NBUF, _BN, _G), jnp.float32),
            pltpu.VMEM((_NBUF, _BN, _G), jnp.float32),
            pltpu.VMEM((_NBUF, _BN, _G), jnp.float32),
            pltpu.VMEM((_NBUF, _BN, _G), jnp.uint8),
            pltpu.VMEM((8, _G), jnp.float32),
            pltpu.VMEM((8, _G), jnp.float32),
            pltpu.SemaphoreType.DMA((_NBUF, 4)),
        ],
        interpret=interpret,
    )(pred_expr, target_expr, obs_mask, mask_u8)
    return out[0, 0]


@jax.jit
def kernel(pred_expr, target_expr, obs_mask, mask_indices):
    mask_u8 = mask_indices.view(jnp.uint8)
    loss = _masked_mse(pred_expr, target_expr, obs_mask, mask_u8)
    return (loss, loss)


# speedup vs baseline: 1.0199x; 1.0199x over previous
import jax
import jax.numpy as jnp
from jax import lax
from jax.experimental import pallas as pl
from jax.experimental.pallas import tpu as pltpu

_N, _G = 100000, 512
_BN = 800
_NBUF = 5
_S = _N // _BN
_OUTER = _S // _NBUF


def _body(pred_hbm, tgt_hbm, obs_hbm, msk_hbm, out_ref,
          pbuf, tbuf, obuf, mbuf, eacc_ref, macc_ref, sems):
    def copies(s, b):
        sl = pl.ds(s * _BN, _BN)
        return (
            pltpu.make_async_copy(pred_hbm.at[sl], pbuf.at[b], sems.at[b, 0]),
            pltpu.make_async_copy(tgt_hbm.at[sl], tbuf.at[b], sems.at[b, 1]),
            pltpu.make_async_copy(obs_hbm.at[sl], obuf.at[b], sems.at[b, 2]),
            pltpu.make_async_copy(msk_hbm.at[sl], mbuf.at[b], sems.at[b, 3]),
        )

    for b in range(_NBUF):
        for c in copies(b, b):
            c.start()

    eacc_ref[...] = jnp.zeros((1, _G), jnp.float32)
    macc_ref[...] = jnp.zeros((1, _G), jnp.float32)

    def outer(g, carry):
        for j in range(_NBUF):
            s = g * _NBUF + j
            for c in copies(s, j):
                c.wait()
            p = pbuf[j]
            t = tbuf[j]
            o = obuf[j]
            mi = mbuf[j]
            m = jnp.where(o > 0.5, mi.astype(jnp.float32), 0.0)
            d = p - t
            e = d * d * m
            eacc_ref[...] += jnp.sum(e, axis=0).reshape(1, _G)
            macc_ref[...] += jnp.sum(m, axis=0).reshape(1, _G)

            @pl.when(s + _NBUF < _S)
            def _():
                for c in copies(s + _NBUF, j):
                    c.start()
        return carry

    lax.fori_loop(0, _OUTER, outer, 0)
    ssum = jnp.sum(eacc_ref[...])
    csum = jnp.sum(macc_ref[...])
    out_ref[0, 0] = ssum / jnp.maximum(csum, 1.0)


def _masked_mse(pred_expr, target_expr, obs_mask, mask_u8, interpret=False):
    out = pl.pallas_call(
        _body,
        in_specs=[pl.BlockSpec(memory_space=pl.ANY)] * 4,
        out_specs=pl.BlockSpec(memory_space=pltpu.SMEM),
        out_shape=jax.ShapeDtypeStruct((1, 1), jnp.float32),
        scratch_shapes=[
            pltpu.VMEM((_NBUF, _BN, _G), jnp.float32),
            pltpu.VMEM((_NBUF, _BN, _G), jnp.float32),
            pltpu.VMEM((_NBUF, _BN, _G), jnp.float32),
            pltpu.VMEM((_NBUF, _BN, _G), jnp.uint8),
            pltpu.VMEM((1, _G), jnp.float32),
            pltpu.VMEM((1, _G), jnp.float32),
            pltpu.SemaphoreType.DMA((_NBUF, 4)),
        ],
        interpret=interpret,
    )(pred_expr, target_expr, obs_mask, mask_u8)
    return out[0, 0]


@jax.jit
def kernel(pred_expr, target_expr, obs_mask, mask_indices):
    mask_u8 = mask_indices.view(jnp.uint8)
    loss = _masked_mse(pred_expr, target_expr, obs_mask, mask_u8)
    return (loss, loss)


# speedup vs baseline: 1.0502x; 1.0296x over previous
import jax
import jax.numpy as jnp
from jax import lax
from jax.experimental import pallas as pl
from jax.experimental.pallas import tpu as pltpu

_N, _G = 100000, 512
_BN = 800
_NBUF = 5
_S = _N // _BN
_OUTER = _S // _NBUF


def _body(pred_hbm, tgt_hbm, obs_hbm, msk_hbm, out_ref,
          pbuf, tbuf, obuf, mbuf, acc_ref, sems):
    def copies(s, b):
        sl = pl.ds(s * _BN, _BN)
        return (
            pltpu.make_async_copy(pred_hbm.at[sl], pbuf.at[b], sems.at[b, 0]),
            pltpu.make_async_copy(tgt_hbm.at[sl], tbuf.at[b], sems.at[b, 1]),
            pltpu.make_async_copy(obs_hbm.at[sl], obuf.at[b], sems.at[b, 2]),
            pltpu.make_async_copy(msk_hbm.at[sl], mbuf.at[b], sems.at[b, 3]),
        )

    for b in range(_NBUF):
        for c in copies(b, b):
            c.start()

    acc_ref[0] = 0.0
    acc_ref[1] = 0.0

    def outer(g, carry):
        for j in range(_NBUF):
            s = g * _NBUF + j
            for c in copies(s, j):
                c.wait()
            p = pbuf[j]
            t = tbuf[j]
            o = obuf[j]
            mi = mbuf[j]
            m = jnp.where(o > 0.5, mi.astype(jnp.float32), 0.0)
            d = p - t
            ssum = jnp.sum(d * d * m)
            csum = jnp.sum(m)
            acc_ref[0] += ssum
            acc_ref[1] += csum

            @pl.when(s + _NBUF < _S)
            def _():
                for c in copies(s + _NBUF, j):
                    c.start()
        return carry

    lax.fori_loop(0, _OUTER, outer, 0)
    out_ref[0, 0] = acc_ref[0] / jnp.maximum(acc_ref[1], 1.0)


def _masked_mse(pred_expr, target_expr, obs_mask, mask_u8, interpret=False):
    out = pl.pallas_call(
        _body,
        in_specs=[pl.BlockSpec(memory_space=pl.ANY)] * 4,
        out_specs=pl.BlockSpec(memory_space=pltpu.SMEM),
        out_shape=jax.ShapeDtypeStruct((1, 1), jnp.float32),
        scratch_shapes=[
            pltpu.VMEM((_NBUF, _BN, _G), jnp.float32),
            pltpu.VMEM((_NBUF, _BN, _G), jnp.float32),
            pltpu.VMEM((_NBUF, _BN, _G), jnp.float32),
            pltpu.VMEM((_NBUF, _BN, _G), jnp.uint8),
            pltpu.SMEM((2,), jnp.float32),
            pltpu.SemaphoreType.DMA((_NBUF, 4)),
        ],
        interpret=interpret,
    )(pred_expr, target_expr, obs_mask, mask_u8)
    return out[0, 0]


@jax.jit
def kernel(pred_expr, target_expr, obs_mask, mask_indices):
    mask_u8 = mask_indices.view(jnp.uint8)
    loss = _masked_mse(pred_expr, target_expr, obs_mask, mask_u8)
    return (loss, loss)
